# baseline (speedup 1.0000x reference)
import functools

import jax
import jax.numpy as jnp
from jax import lax
from jax.experimental import pallas as pl
from jax.experimental.pallas import tpu as pltpu
from jax.experimental.pallas import tpu_sc as plsc

K = 1024
D = 256
H = 8
BLK = 256
N_TOK = 4096
TAU = 1.0
COMMIT = 0.25
ENT_RATIO = 0.1
ENT_TEMP = 0.01

NW = 32
TPW = N_TOK // NW
CT = 16
L = 16


def _dist_ent_block(x_ref, cb_ref, d_ref, ent_ref, avgp_acc, sacc,
                    *, n_blocks):
    i = pl.program_id(0)

    @pl.when(i == 0)
    def _init():
        avgp_acc[...] = jnp.zeros_like(avgp_acc)
        sacc[0] = 0.0

    x = x_ref[...]
    cb = cb_ref[...]
    ab = lax.dot_general(x, cb, (((1,), (1,)), ((), ())),
                         preferred_element_type=jnp.float32)
    x2 = jnp.sum(x * x, axis=1, keepdims=True)
    b2 = jnp.sum(cb * cb, axis=1)[None, :]
    d = x2 - 2.0 * ab + b2
    d_ref[...] = d

    a = d * (-1.0 / ENT_TEMP)
    m = jnp.max(a, axis=1, keepdims=True)
    e = jnp.exp(a - m)
    z = jnp.sum(e, axis=1, keepdims=True)
    p = e / z
    s_ent = jnp.log(z[:, 0]) - jnp.sum(e * (a - m), axis=1) / z[:, 0]
    avgp_acc[...] += jnp.sum(p, axis=0, keepdims=True)
    sacc[0] += jnp.sum(s_ent)

    @pl.when(i == n_blocks - 1)
    def _fin():
        navg = 1.0 / N_TOK
        avg_p = avgp_acc[...] * navg
        avg_ent = -jnp.sum(avg_p * jnp.log(avg_p + 1e-5))
        ent_ref[...] = jnp.reshape(
            ENT_RATIO * (sacc[0] * navg - avg_ent), (1, 1))


def _dist_ent(x2d, cb):
    n_blocks = N_TOK // BLK
    kern = functools.partial(_dist_ent_block, n_blocks=n_blocks)
    return pl.pallas_call(
        kern,
        grid=(n_blocks,),
        in_specs=[
            pl.BlockSpec((BLK, D), lambda i: (i, 0)),
            pl.BlockSpec((K, D), lambda i: (0, 0)),
        ],
        out_specs=[
            pl.BlockSpec((BLK, K), lambda i: (i, 0)),
            pl.BlockSpec((1, 1), lambda i: (0, 0)),
        ],
        out_shape=[
            jax.ShapeDtypeStruct((N_TOK, K), jnp.float32),
            jax.ShapeDtypeStruct((1, 1), jnp.float32),
        ],
        scratch_shapes=[
            pltpu.VMEM((1, K), jnp.float32),
            pltpu.SMEM((2,), jnp.float32),
        ],
    )(x2d, cb)


def _topk_body(dist_hbm, ti_hbm, tw_hbm, dbuf, cvals, cidx, tibuf, twbuf):
    wid = lax.axis_index("s") * 2 + lax.axis_index("c")
    lane = lax.iota(jnp.int32, L)
    inf_v = jnp.full((L,), jnp.inf, jnp.float32)

    def chunk_body(ci, _):
        tok0 = wid * TPW + ci * CT
        pltpu.sync_copy(dist_hbm.at[pl.ds(tok0 * K, CT * K)], dbuf)

        def token_body(t, _):
            base = t * K

            def pmin_body(j, pm):
                v = dbuf[pl.ds(base + j * L, L)]
                return jnp.minimum(pm, v)
            pm = lax.fori_loop(0, K // L, pmin_body, inf_v)

            sk, _sv = plsc.sort_key_val(pm, lane)
            thr = jnp.broadcast_to(sk[H - 1], (L,))

            def collect_body(j, cnt):
                v = dbuf[pl.ds(base + j * L, L)]
                msk = v <= thr
                c = plsc.cumsum(msk.astype(jnp.int32))
                pos = c + (cnt - 1)
                plsc.store_scatter(cvals, [pos], v, mask=msk)
                plsc.store_scatter(cidx, [pos], lane + j * L, mask=msk)
                return cnt + plsc.all_reduce_population_count(msk)[0]
            cnt = lax.fori_loop(0, K // L, collect_body, jnp.int32(0))
            plsc.store_scatter(cvals, [lane + cnt], inf_v)
            nv = (cnt + 15) // 16

            tvals = inf_v
            tidx = jnp.zeros((L,), jnp.int32)
            for r in range(H):
                def min_body(j, mv):
                    return jnp.minimum(mv, cvals[pl.ds(j * L, L)])
                mv = lax.fori_loop(0, nv, min_body, inf_v)
                s = jnp.min(mv)
                s_v = jnp.broadcast_to(s, (L,))

                def pos_body(j, pv):
                    hit = cvals[pl.ds(j * L, L)] == s_v
                    return jnp.minimum(
                        pv, jnp.where(hit, lane + j * L, jnp.int32(2048)))
                pv = lax.fori_loop(0, nv, pos_body,
                                   jnp.full((L,), 2048, jnp.int32))
                p_v = jnp.broadcast_to(jnp.min(pv), (L,))
                oi = plsc.load_gather(cidx, [p_v])
                tvals = jnp.where(lane == r, s_v, tvals)
                tidx = jnp.where(lane == r, oi, tidx)
                plsc.store_scatter(cvals, [p_v], inf_v, mask=lane == 0)

            d0 = jnp.broadcast_to(tvals[0], (L,))
            e = jnp.where(lane < H, jnp.exp(d0 - tvals), 0.0)
            tw = e / jnp.broadcast_to(jnp.sum(e), (L,))
            plsc.store_scatter(tibuf, [t * H + lane], tidx, mask=lane < H)
            plsc.store_scatter(twbuf, [t * H + lane], tw, mask=lane < H)
            return 0

        lax.fori_loop(0, CT, token_body, 0)
        pltpu.sync_copy(tibuf, ti_hbm.at[pl.ds(tok0 * H, CT * H)])
        pltpu.sync_copy(twbuf, tw_hbm.at[pl.ds(tok0 * H, CT * H)])
        return 0

    lax.fori_loop(0, TPW // CT, chunk_body, 0)


def _topk_sc(dist_flat):
    mesh = plsc.VectorSubcoreMesh(core_axis_name="c", subcore_axis_name="s")
    f = functools.partial(
        pl.kernel,
        mesh=mesh,
        compiler_params=pltpu.CompilerParams(needs_layout_passes=False),
        out_type=[
            jax.ShapeDtypeStruct((N_TOK * H,), jnp.int32),
            jax.ShapeDtypeStruct((N_TOK * H,), jnp.float32),
        ],
        scratch_types=[
            pltpu.VMEM((CT * K,), jnp.float32),
            pltpu.VMEM((K + L,), jnp.float32),
            pltpu.VMEM((K + L,), jnp.int32),
            pltpu.VMEM((CT * H,), jnp.int32),
            pltpu.VMEM((CT * H,), jnp.float32),
        ],
    )(_topk_body)
    return f(dist_flat)


def _finish_block(x_ref, cb_ref, ti_ref, tw_ref, ent_ref,
                  enc_ref, q_ref, loss_ref, sacc, *, n_blocks):
    i = pl.program_id(0)

    @pl.when(i == 0)
    def _init():
        sacc[0] = 0.0

    x = x_ref[...]
    cb = cb_ref[...]
    ti = ti_ref[...]
    tw = tw_ref[...]
    iota_k = lax.broadcasted_iota(jnp.int32, (BLK, K), 1)
    enc = jnp.zeros((BLK, K), jnp.float32)
    for h in range(H):
        enc += jnp.where(iota_k == ti[:, h:h + 1], tw[:, h:h + 1], 0.0)
    enc_ref[...] = enc
    q = lax.dot_general(enc, cb, (((1,), (0,)), ((), ())),
                        preferred_element_type=jnp.float32)
    q_ref[...] = q
    r = q - x
    sacc[0] += jnp.sum(r * r)

    @pl.when(i == n_blocks - 1)
    def _fin():
        mse = sacc[0] * (1.0 / (N_TOK * D))
        loss_ref[...] = jnp.reshape(
            (1.0 + COMMIT) * mse + ent_ref[0, 0], (1, 1))


def _finish(x2d, cb, ti, tw, ent):
    n_blocks = N_TOK // BLK
    kern = functools.partial(_finish_block, n_blocks=n_blocks)
    return pl.pallas_call(
        kern,
        grid=(n_blocks,),
        in_specs=[
            pl.BlockSpec((BLK, D), lambda i: (i, 0)),
            pl.BlockSpec((K, D), lambda i: (0, 0)),
            pl.BlockSpec((BLK, H), lambda i: (i, 0)),
            pl.BlockSpec((BLK, H), lambda i: (i, 0)),
            pl.BlockSpec((1, 1), lambda i: (0, 0)),
        ],
        out_specs=[
            pl.BlockSpec((BLK, K), lambda i: (i, 0)),
            pl.BlockSpec((BLK, D), lambda i: (i, 0)),
            pl.BlockSpec((1, 1), lambda i: (0, 0)),
        ],
        out_shape=[
            jax.ShapeDtypeStruct((N_TOK, K), jnp.float32),
            jax.ShapeDtypeStruct((N_TOK, D), jnp.float32),
            jax.ShapeDtypeStruct((1, 1), jnp.float32),
        ],
        scratch_shapes=[
            pltpu.SMEM((2,), jnp.float32),
        ],
    )(x2d, cb, ti, tw, ent)


@jax.jit
def _vq(x2d, cb):
    dist, ent = _dist_ent(x2d, cb)
    ti_f, tw_f = _topk_sc(dist.reshape(-1))
    ti = ti_f.reshape(N_TOK, H)
    tw = tw_f.reshape(N_TOK, H)
    enc, q, loss = _finish(x2d, cb, ti, tw, ent)
    return q, loss, ti, tw, enc


def kernel(x, codebook):
    b, t, d = x.shape
    x2d = x.reshape(b * t, d)
    q, loss, ti, tw, enc = _vq(x2d, codebook)
    return (q.reshape(b, t, d), loss[0, 0], ti.reshape(b, t, H),
            tw.reshape(b, t, H), enc.reshape(b, t, K))


# speedup vs baseline: 1.6361x; 1.6361x over previous
import functools

import jax
import jax.numpy as jnp
from jax import lax
from jax.experimental import pallas as pl
from jax.experimental.pallas import tpu as pltpu
from jax.experimental.pallas import tpu_sc as plsc

K = 1024
D = 256
H = 8
BLK = 256
N_TOK = 4096
TAU = 1.0
COMMIT = 0.25
ENT_RATIO = 0.1
ENT_TEMP = 0.01

NW = 32
TPW = N_TOK // NW
CT = 16
L = 16


def _dist_ent_block(x_ref, cb_ref, d_ref, ent_ref, avgp_acc, sacc,
                    *, n_blocks):
    i = pl.program_id(0)

    @pl.when(i == 0)
    def _init():
        avgp_acc[...] = jnp.zeros_like(avgp_acc)
        sacc[0] = 0.0

    x = x_ref[...]
    cb = cb_ref[...]
    ab = lax.dot_general(x, cb, (((1,), (1,)), ((), ())),
                         preferred_element_type=jnp.float32)
    x2 = jnp.sum(x * x, axis=1, keepdims=True)
    b2 = jnp.sum(cb * cb, axis=1)[None, :]
    d = x2 - 2.0 * ab + b2
    d_ref[...] = d

    a = d * (-1.0 / ENT_TEMP)
    m = jnp.max(a, axis=1, keepdims=True)
    e = jnp.exp(a - m)
    z = jnp.sum(e, axis=1, keepdims=True)
    p = e / z
    s_ent = jnp.log(z[:, 0]) - jnp.sum(e * (a - m), axis=1) / z[:, 0]
    avgp_acc[...] += jnp.sum(p, axis=0, keepdims=True)
    sacc[0] += jnp.sum(s_ent)

    @pl.when(i == n_blocks - 1)
    def _fin():
        navg = 1.0 / N_TOK
        avg_p = avgp_acc[...] * navg
        avg_ent = -jnp.sum(avg_p * jnp.log(avg_p + 1e-5))
        ent_ref[...] = jnp.reshape(
            ENT_RATIO * (sacc[0] * navg - avg_ent), (1, 1))


def _dist_ent(x2d, cb):
    n_blocks = N_TOK // BLK
    kern = functools.partial(_dist_ent_block, n_blocks=n_blocks)
    return pl.pallas_call(
        kern,
        grid=(n_blocks,),
        in_specs=[
            pl.BlockSpec((BLK, D), lambda i: (i, 0)),
            pl.BlockSpec((K, D), lambda i: (0, 0)),
        ],
        out_specs=[
            pl.BlockSpec((BLK, K), lambda i: (i, 0)),
            pl.BlockSpec((1, 1), lambda i: (0, 0)),
        ],
        out_shape=[
            jax.ShapeDtypeStruct((N_TOK, K), jnp.float32),
            jax.ShapeDtypeStruct((1, 1), jnp.float32),
        ],
        scratch_shapes=[
            pltpu.VMEM((1, K), jnp.float32),
            pltpu.SMEM((2,), jnp.float32),
        ],
    )(x2d, cb)


def _topk_body(dist_hbm, ti_hbm, tw_hbm, dbuf, cidx, cvals, tibuf, twbuf, sem):
    wid = lax.axis_index("s") * 2 + lax.axis_index("c")
    lane = lax.iota(jnp.int32, L)
    inf_v = jnp.full((L,), jnp.inf, jnp.float32)
    n_chunks = TPW // CT

    def chunk_start(ci, buf):
        tok0 = wid * TPW + ci * CT
        pltpu.async_copy(dist_hbm.at[pl.ds(tok0, CT)], dbuf.at[buf], sem)

    chunk_start(0, 0)

    def chunk_body(ci, _):
        tok0 = wid * TPW + ci * CT
        b = ci % 2

        @pl.when(ci < n_chunks - 1)
        def _prefetch():
            chunk_start(ci + 1, (ci + 1) % 2)

        pltpu.make_async_copy(
            dist_hbm.at[pl.ds(tok0, CT)], dbuf.at[b], sem).wait()

        def token_body(t, _):
            def pmin_body(j, pm):
                c0 = j * (4 * L)
                v0 = dbuf[b, t, pl.ds(c0, L)]
                v1 = dbuf[b, t, pl.ds(c0 + L, L)]
                v2 = dbuf[b, t, pl.ds(c0 + 2 * L, L)]
                v3 = dbuf[b, t, pl.ds(c0 + 3 * L, L)]
                m = jnp.minimum(jnp.minimum(v0, v1), jnp.minimum(v2, v3))
                return jnp.minimum(pm, m)
            pm = lax.fori_loop(0, K // (4 * L), pmin_body, inf_v)

            sk, _sv = plsc.sort_key_val(pm, lane)
            thr = jnp.broadcast_to(sk[H - 1], (L,))

            def collect_body(j, cnt):
                c0 = j * (4 * L)
                for u in range(4):
                    v = dbuf[b, t, pl.ds(c0 + u * L, L)]
                    msk = v <= thr
                    plsc.store_compressed(cidx.at[pl.ds(cnt, L)],
                                          lane + (c0 + u * L), mask=msk)
                    cnt = cnt + plsc.all_reduce_population_count(msk)[0]
                return cnt
            cnt = lax.fori_loop(0, K // (4 * L), collect_body, jnp.int32(0))

            t_v = jnp.broadcast_to(t, (L,))

            def fast_path(_):
                iv = cidx[pl.ds(0, L)]
                iv = jnp.where(lane < cnt, iv, jnp.int32(K - 1))
                v = plsc.load_gather(dbuf, [jnp.broadcast_to(b, (L,)),
                                            t_v, iv])
                v = jnp.where(lane < cnt, v, jnp.inf)
                sk2, _ = plsc.sort_key_val(v, lane)
                used = lane >= cnt
                tidx = jnp.zeros((L,), jnp.int32)
                for r in range(H):
                    srv = jnp.broadcast_to(sk2[r], (L,))
                    hit = jnp.logical_and(v == srv, jnp.logical_not(used))
                    p_v = plsc.all_reduce_ffs(hit)
                    used = jnp.logical_or(used, lane == p_v)
                    oi = iv.at[p_v].get(mode="promise_in_bounds")
                    tidx = jnp.where(lane == r, oi, tidx)
                return sk2, tidx

            def gen_path(_):
                plsc.store_scatter(cidx, [lane + cnt],
                                   jnp.full((L,), K - 1, jnp.int32))
                nv = (cnt + 15) // 16

                def fill_body(j, _c):
                    iv = cidx[pl.ds(j * L, L)]
                    v = plsc.load_gather(dbuf, [jnp.broadcast_to(b, (L,)),
                                                t_v, iv])
                    off = jnp.where(lane + j * L < cnt, 0.0, jnp.inf)
                    cvals[pl.ds(j * L, L)] = v + off
                    return 0
                lax.fori_loop(0, nv, fill_body, 0)

                tvals = inf_v
                tidx = jnp.zeros((L,), jnp.int32)
                for r in range(H):
                    def min_body(j, mv):
                        return jnp.minimum(mv, cvals[pl.ds(j * L, L)])
                    mv = lax.fori_loop(0, nv, min_body, inf_v)
                    s_v = jnp.broadcast_to(jnp.min(mv), (L,))

                    def pos_body(j, pv):
                        hit = cvals[pl.ds(j * L, L)] == s_v
                        return jnp.minimum(
                            pv, jnp.where(hit, lane + j * L, jnp.int32(2048)))
                    pv = lax.fori_loop(0, nv, pos_body,
                                       jnp.full((L,), 2048, jnp.int32))
                    p_v = jnp.broadcast_to(jnp.min(pv), (L,))
                    oi = plsc.load_gather(cidx, [p_v])
                    tvals = jnp.where(lane == r, s_v, tvals)
                    tidx = jnp.where(lane == r, oi, tidx)
                    plsc.store_scatter(cvals, [p_v], inf_v, mask=lane == 0)
                return tvals, tidx

            tvals, tidx = lax.cond(cnt <= L, fast_path, gen_path, 0)

            d0 = jnp.broadcast_to(tvals[0], (L,))
            e = jnp.where(lane < H, jnp.exp(d0 - tvals), 0.0)
            tw = e / jnp.broadcast_to(jnp.sum(e), (L,))
            plsc.store_scatter(tibuf, [t_v, lane], tidx, mask=lane < H)
            plsc.store_scatter(twbuf, [t_v, lane], tw, mask=lane < H)
            return 0

        lax.fori_loop(0, CT, token_body, 0)
        pltpu.sync_copy(tibuf, ti_hbm.at[pl.ds(tok0, CT)])
        pltpu.sync_copy(twbuf, tw_hbm.at[pl.ds(tok0, CT)])
        return 0

    lax.fori_loop(0, n_chunks, chunk_body, 0)


def _topk_sc(dist):
    mesh = plsc.VectorSubcoreMesh(core_axis_name="c", subcore_axis_name="s")
    f = functools.partial(
        pl.kernel,
        mesh=mesh,
        compiler_params=pltpu.CompilerParams(needs_layout_passes=False),
        out_type=[
            jax.ShapeDtypeStruct((N_TOK, H), jnp.int32),
            jax.ShapeDtypeStruct((N_TOK, H), jnp.float32),
        ],
        scratch_types=[
            pltpu.VMEM((2, CT, K), jnp.float32),
            pltpu.VMEM((K + L,), jnp.int32),
            pltpu.VMEM((K + L,), jnp.float32),
            pltpu.VMEM((CT, H), jnp.int32),
            pltpu.VMEM((CT, H), jnp.float32),
            pltpu.SemaphoreType.DMA,
        ],
    )(_topk_body)
    return f(dist)


def _finish_block(x_ref, cb_ref, ti_ref, tw_ref, ent_ref,
                  enc_ref, q_ref, loss_ref, sacc, *, n_blocks):
    i = pl.program_id(0)

    @pl.when(i == 0)
    def _init():
        sacc[0] = 0.0

    x = x_ref[...]
    cb = cb_ref[...]
    ti = ti_ref[...]
    tw = tw_ref[...]
    iota_k = lax.broadcasted_iota(jnp.int32, (BLK, K), 1)
    enc = jnp.zeros((BLK, K), jnp.float32)
    for h in range(H):
        enc += jnp.where(iota_k == ti[:, h:h + 1], tw[:, h:h + 1], 0.0)
    enc_ref[...] = enc
    q = lax.dot_general(enc, cb, (((1,), (0,)), ((), ())),
                        preferred_element_type=jnp.float32)
    q_ref[...] = q
    r = q - x
    sacc[0] += jnp.sum(r * r)

    @pl.when(i == n_blocks - 1)
    def _fin():
        mse = sacc[0] * (1.0 / (N_TOK * D))
        loss_ref[...] = jnp.reshape(
            (1.0 + COMMIT) * mse + ent_ref[0, 0], (1, 1))


def _finish(x2d, cb, ti, tw, ent):
    n_blocks = N_TOK // BLK
    kern = functools.partial(_finish_block, n_blocks=n_blocks)
    return pl.pallas_call(
        kern,
        grid=(n_blocks,),
        in_specs=[
            pl.BlockSpec((BLK, D), lambda i: (i, 0)),
            pl.BlockSpec((K, D), lambda i: (0, 0)),
            pl.BlockSpec((BLK, H), lambda i: (i, 0)),
            pl.BlockSpec((BLK, H), lambda i: (i, 0)),
            pl.BlockSpec((1, 1), lambda i: (0, 0)),
        ],
        out_specs=[
            pl.BlockSpec((BLK, K), lambda i: (i, 0)),
            pl.BlockSpec((BLK, D), lambda i: (i, 0)),
            pl.BlockSpec((1, 1), lambda i: (0, 0)),
        ],
        out_shape=[
            jax.ShapeDtypeStruct((N_TOK, K), jnp.float32),
            jax.ShapeDtypeStruct((N_TOK, D), jnp.float32),
            jax.ShapeDtypeStruct((1, 1), jnp.float32),
        ],
        scratch_shapes=[
            pltpu.SMEM((2,), jnp.float32),
        ],
    )(x2d, cb, ti, tw, ent)


@jax.jit
def _vq(x2d, cb):
    dist, ent = _dist_ent(x2d, cb)
    ti, tw = _topk_sc(dist)
    enc, q, loss = _finish(x2d, cb, ti, tw, ent)
    return q, loss, ti, tw, enc


def kernel(x, codebook):
    b, t, d = x.shape
    x2d = x.reshape(b * t, d)
    q, loss, ti, tw, enc = _vq(x2d, codebook)
    return (q.reshape(b, t, d), loss[0, 0], ti.reshape(b, t, H),
            tw.reshape(b, t, H), enc.reshape(b, t, K))
